# baseline (speedup 1.0000x reference)
import jax
import jax.numpy as jnp
from jax import lax
from jax.experimental import pallas as pl
from jax.experimental.pallas import tpu as pltpu
from jax.experimental.pallas import tpu_sc as plsc

D_EMBED = 64
BATCH = 16384
N_NEG = 5

_NC = 2
_NS = 16
_NW = _NC * _NS
_L = 128
_NBUF = 4

_PC_CH = BATCH // _NW // _L
_CX_CH = BATCH * (1 + N_NEG) // _NW // _L


def _body(wc_hbm, wx_hbm, pc_idx, cx_idx, out_pc, out_cx,
          idx_v, rows_v, gsem, wsem):
  wid = lax.axis_index("s") * _NC + lax.axis_index("c")

  pltpu.sync_copy(pc_idx.at[pl.ds(wid * _PC_CH, _PC_CH)],
                  idx_v.at[pl.ds(0, _PC_CH)])
  pltpu.sync_copy(cx_idx.at[pl.ds(wid * _CX_CH, _CX_CH)],
                  idx_v.at[pl.ds(_PC_CH, _CX_CH)])

  def phase(tbl, out, idx_off, n_ch):
    cbase = wid * n_ch

    def group(g):
      cps = []
      for b in range(_NBUF):
        cp = pltpu.make_async_copy(
            tbl.at[idx_v.at[idx_off + g + b]], rows_v.at[b], gsem.at[b])
        cp.start()
        cps.append(cp)
      wps = []
      for b in range(_NBUF):
        cps[b].wait()
        wp = pltpu.make_async_copy(
            rows_v.at[b], out.at[pl.ds((cbase + g + b) * _L, _L)],
            wsem.at[b])
        wp.start()
        wps.append(wp)
      for b in range(_NBUF):
        wps[b].wait()

    if n_ch <= _NBUF:
      group(0)
    else:
      lax.fori_loop(0, n_ch // _NBUF, lambda i, c: (group(i * _NBUF), c)[1],
                    0)

  phase(wc_hbm, out_pc, 0, _PC_CH)
  phase(wx_hbm, out_cx, _PC_CH, _CX_CH)


@jax.jit
def _gather(W_center, W_context, pc_idx, cx_idx):
  run = pl.kernel(
      _body,
      out_type=(
          jax.ShapeDtypeStruct((BATCH, D_EMBED), jnp.float32),
          jax.ShapeDtypeStruct((BATCH * (1 + N_NEG), D_EMBED), jnp.float32),
      ),
      mesh=plsc.VectorSubcoreMesh(core_axis_name="c", subcore_axis_name="s"),
      scratch_types=[
          pltpu.VMEM((_PC_CH + _CX_CH, _L), jnp.int32),
          pltpu.VMEM((_NBUF, _L, D_EMBED), jnp.float32),
          pltpu.SemaphoreType.DMA((_NBUF,)),
          pltpu.SemaphoreType.DMA((_NBUF,)),
      ],
      compiler_params=pltpu.CompilerParams(
          use_tc_tiling_on_sc=False,
          disable_bounds_checks=True,
          disable_semaphore_checks=True,
      ),
  )
  out_pc, out_cx = run(W_center, W_context, pc_idx, cx_idx)
  px = out_cx[:BATCH]
  nx = out_cx[BATCH:].reshape(BATCH, N_NEG, D_EMBED)
  return out_pc, px, nx


def kernel(W_center, W_context, pos_center, pos_context, neg_context):
  pc_idx = pos_center.astype(jnp.int32).reshape(-1, _L)
  cx_idx = jnp.concatenate(
      [pos_context.astype(jnp.int32),
       neg_context.astype(jnp.int32).reshape(-1)]).reshape(-1, _L)
  return _gather(W_center, W_context, pc_idx, cx_idx)


# speedup vs baseline: 1.0653x; 1.0653x over previous
import jax
import jax.numpy as jnp
from jax import lax
from jax.experimental import pallas as pl
from jax.experimental.pallas import tpu as pltpu
from jax.experimental.pallas import tpu_sc as plsc

D_EMBED = 64
BATCH = 16384
N_NEG = 5

_NC = 2
_NS = 16
_NW = _NC * _NS
_L = 128
_NBUF = 12

_PC_N = BATCH // _NW
_NX_N = BATCH * N_NEG // _NW
_PC_CH = _PC_N // _L
_NX_CH = _NX_N // _L


def _body(wc_hbm, wx_hbm, pc_idx, px_idx, nx_idx,
          out_pc, out_px, out_nx,
          idx_v, rows_v, gsem, wsem):
  wid = lax.axis_index("s") * _NC + lax.axis_index("c")

  pltpu.sync_copy(pc_idx.at[pl.ds(wid * _PC_N, _PC_N)],
                  idx_v.at[pl.ds(0, _PC_N)])
  pltpu.sync_copy(px_idx.at[pl.ds(wid * _PC_N, _PC_N)],
                  idx_v.at[pl.ds(_PC_N, _PC_N)])
  pltpu.sync_copy(nx_idx.at[pl.ds(wid * _NX_N, _NX_N)],
                  idx_v.at[pl.ds(2 * _PC_N, _NX_N)])

  items = []
  for j in range(_PC_CH):
    items.append((wc_hbm, j * _L, out_pc, (wid * _PC_CH + j) * _L))
  for j in range(_PC_CH):
    items.append((wx_hbm, _PC_N + j * _L, out_px, (wid * _PC_CH + j) * _L))
  for j in range(_NX_CH):
    items.append((wx_hbm, 2 * _PC_N + j * _L, out_nx,
                  (wid * _NX_CH + j) * _L))

  n = len(items)
  g = [None] * n
  w = [None] * n

  def start_gather(j):
    tbl, ioff, _, _ = items[j]
    b = j % _NBUF
    g[j] = pltpu.make_async_copy(tbl.at[idx_v.at[pl.ds(ioff, _L)]],
                                 rows_v.at[b], gsem.at[b])
    g[j].start()

  def start_write(j):
    _, _, out, rbase = items[j]
    b = j % _NBUF
    w[j] = pltpu.make_async_copy(rows_v.at[b], out.at[pl.ds(rbase, _L)],
                                 wsem.at[b])
    w[j].start()

  for j in range(n + _NBUF - 1):
    if j < n:
      if j >= _NBUF:
        w[j - _NBUF].wait()
      start_gather(j)
    k = j - (_NBUF - 1)
    if 0 <= k < n:
      g[k].wait()
      start_write(k)

  for j in range(max(0, n - _NBUF), n):
    w[j].wait()


@jax.jit
def _gather(W_center, W_context, pc_idx, px_idx, nx_idx):
  run = pl.kernel(
      _body,
      out_type=(
          jax.ShapeDtypeStruct((BATCH, D_EMBED), jnp.float32),
          jax.ShapeDtypeStruct((BATCH, D_EMBED), jnp.float32),
          jax.ShapeDtypeStruct((BATCH * N_NEG, D_EMBED), jnp.float32),
      ),
      mesh=plsc.VectorSubcoreMesh(core_axis_name="c", subcore_axis_name="s"),
      scratch_types=[
          pltpu.VMEM((2 * _PC_N + _NX_N,), jnp.int32),
          pltpu.VMEM((_NBUF, _L, D_EMBED), jnp.float32),
          pltpu.SemaphoreType.DMA((_NBUF,)),
          pltpu.SemaphoreType.DMA((_NBUF,)),
      ],
      compiler_params=pltpu.CompilerParams(use_tc_tiling_on_sc=False),
  )
  return run(W_center, W_context, pc_idx, px_idx, nx_idx)


def kernel(W_center, W_context, pos_center, pos_context, neg_context):
  pc, px, nx = _gather(W_center, W_context,
                       pos_center.astype(jnp.int32),
                       pos_context.astype(jnp.int32),
                       neg_context.astype(jnp.int32).reshape(-1))
  return pc, px, nx.reshape(BATCH, N_NEG, D_EMBED)


# speedup vs baseline: 1.1247x; 1.0558x over previous
import jax
import jax.numpy as jnp
from jax import lax
from jax.experimental import pallas as pl
from jax.experimental.pallas import tpu as pltpu
from jax.experimental.pallas import tpu_sc as plsc

D_EMBED = 64
D_PAD = 128
BATCH = 16384
N_NEG = 5

_NC = 2
_NS = 16
_NW = _NC * _NS
_L = 128
_NBUF = 6

_PC_CH = BATCH // _NW // _L
_NX_CH = BATCH * N_NEG // _NW // _L


def _run_ring(items, idx_v, rows_v, gsem, wsem):
  n = len(items)
  g = [None] * n
  w = [None] * n

  def start_gather(j):
    tbl, irow, _, _ = items[j]
    b = j % _NBUF
    g[j] = pltpu.make_async_copy(tbl.at[idx_v.at[irow]], rows_v.at[b],
                                 gsem.at[b])
    g[j].start()

  def start_write(j):
    _, _, out, rbase = items[j]
    b = j % _NBUF
    w[j] = pltpu.make_async_copy(rows_v.at[b], out.at[pl.ds(rbase, _L)],
                                 wsem.at[b])
    w[j].start()

  for j in range(n + _NBUF - 1):
    if j < n:
      if j >= _NBUF:
        w[j - _NBUF].wait()
      start_gather(j)
    k = j - (_NBUF - 1)
    if 0 <= k < n:
      g[k].wait()
      start_write(k)

  for j in range(max(0, n - _NBUF), n):
    w[j].wait()


def _center_body(w_hbm, pc_idx, out_pc, idx_v, rows_v, gsem, wsem):
  wid = lax.axis_index("s") * _NC + lax.axis_index("c")
  pltpu.sync_copy(pc_idx.at[pl.ds(wid * _PC_CH, _PC_CH)],
                  idx_v.at[pl.ds(0, _PC_CH)])
  items = [(w_hbm, j, out_pc, (wid * _PC_CH + j) * _L)
           for j in range(_PC_CH)]
  _run_ring(items, idx_v, rows_v, gsem, wsem)


def _context_body(w_hbm, px_idx, nx_idx, out_px, out_nx,
                  idx_v, rows_v, gsem, wsem):
  wid = lax.axis_index("s") * _NC + lax.axis_index("c")
  pltpu.sync_copy(px_idx.at[pl.ds(wid * _PC_CH, _PC_CH)],
                  idx_v.at[pl.ds(0, _PC_CH)])
  pltpu.sync_copy(nx_idx.at[pl.ds(wid * _NX_CH, _NX_CH)],
                  idx_v.at[pl.ds(_PC_CH, _NX_CH)])
  items = [(w_hbm, j, out_px, (wid * _PC_CH + j) * _L)
           for j in range(_PC_CH)]
  items += [(w_hbm, _PC_CH + j, out_nx, (wid * _NX_CH + j) * _L)
            for j in range(_NX_CH)]
  _run_ring(items, idx_v, rows_v, gsem, wsem)


def _scratch():
  return [
      pltpu.VMEM((_PC_CH + _NX_CH, _L), jnp.int32),
      pltpu.VMEM((_NBUF, _L, D_PAD), jnp.float32),
      pltpu.SemaphoreType.DMA((_NBUF,)),
      pltpu.SemaphoreType.DMA((_NBUF,)),
  ]


_MESH = dict(core_axis_name="c", subcore_axis_name="s")


@jax.jit
def _gather(W_center, W_context, pc_idx, px_idx, nx_idx):
  wc = jnp.pad(W_center, ((0, 0), (0, D_PAD - D_EMBED)))
  wx = jnp.pad(W_context, ((0, 0), (0, D_PAD - D_EMBED)))
  params = pltpu.CompilerParams(use_tc_tiling_on_sc=False)
  px, nx = pl.kernel(
      _context_body,
      out_type=(
          jax.ShapeDtypeStruct((BATCH, D_PAD), jnp.float32),
          jax.ShapeDtypeStruct((BATCH * N_NEG, D_PAD), jnp.float32),
      ),
      mesh=plsc.VectorSubcoreMesh(**_MESH),
      scratch_types=_scratch(),
      compiler_params=params,
  )(wx, px_idx, nx_idx)
  pc = pl.kernel(
      _center_body,
      out_type=jax.ShapeDtypeStruct((BATCH, D_PAD), jnp.float32),
      mesh=plsc.VectorSubcoreMesh(**_MESH),
      scratch_types=_scratch(),
      compiler_params=params,
  )(wc, pc_idx)
  return (pc[:, :D_EMBED], px[:, :D_EMBED],
          nx[:, :D_EMBED].reshape(BATCH, N_NEG, D_EMBED))


def kernel(W_center, W_context, pos_center, pos_context, neg_context):
  pc_idx = pos_center.astype(jnp.int32).reshape(-1, _L)
  px_idx = pos_context.astype(jnp.int32).reshape(-1, _L)
  nx_idx = neg_context.astype(jnp.int32).reshape(-1, _L)
  return _gather(W_center, W_context, pc_idx, px_idx, nx_idx)
